# baseline (speedup 1.0000x reference)
import functools

import jax
import jax.numpy as jnp
from jax import lax
from jax.experimental import pallas as pl
from jax.experimental.pallas import tpu as pltpu
from jax.experimental.pallas import tpu_sc as plsc

_LANES = 16


def _mm_body(x_ref, wlo_ref, whi_ref, blo_ref, bhi_ref, y32_ref):
    s_lo = (
        jnp.dot(x_ref[...], wlo_ref[...], preferred_element_type=jnp.float32)
        + blo_ref[...]
    )
    s_hi = (
        jnp.dot(x_ref[...], whi_ref[...], preferred_element_type=jnp.float32)
        + bhi_ref[...]
    )
    u_lo = lax.bitcast_convert_type(
        s_lo.astype(jnp.bfloat16), jnp.uint16).astype(jnp.int32)
    u_hi = lax.bitcast_convert_type(
        s_hi.astype(jnp.bfloat16), jnp.uint16).astype(jnp.int32)
    y32_ref[...] = u_lo | (u_hi << 16)


def _matmul_bias_packed(x, w_lo, w_hi, b_lo, b_hi):
    n, d_in = x.shape
    dw = w_lo.shape[1]
    bn = 2000
    assert n % bn == 0
    return pl.pallas_call(
        _mm_body,
        grid=(n // bn,),
        in_specs=[
            pl.BlockSpec((bn, d_in), lambda i: (i, 0)),
            pl.BlockSpec((d_in, dw), lambda i: (0, 0)),
            pl.BlockSpec((d_in, dw), lambda i: (0, 0)),
            pl.BlockSpec((1, dw), lambda i: (0, 0)),
            pl.BlockSpec((1, dw), lambda i: (0, 0)),
        ],
        out_specs=pl.BlockSpec((bn, dw), lambda i: (i, 0)),
        out_shape=jax.ShapeDtypeStruct((n, dw), jnp.int32),
    )(x, w_lo, w_hi, b_lo, b_hi)


_C = 40
_NBUF = 3


def _gather_add_sc(y32, idx2, e):
    n, dw = y32.shape
    d = 2 * dw
    info = plsc.get_sparse_core_info()
    nw = info.num_cores * info.num_subcores
    assert e % (nw * _C) == 0
    nch = e // (nw * _C)
    n_outer = (nch + _NBUF - 1) // _NBUF
    mesh = plsc.VectorSubcoreMesh(core_axis_name="c", subcore_axis_name="s")

    @functools.partial(
        pl.kernel,
        mesh=mesh,
        out_type=jax.ShapeDtypeStruct((e, d), jnp.float32),
        scratch_types=(
            [pltpu.VMEM((2 * _C, dw), jnp.int32) for _ in range(_NBUF)]
            + [pltpu.VMEM((_C, d), jnp.float32) for _ in range(_NBUF)]
            + [pltpu.VMEM((2 * _C,), jnp.int32) for _ in range(_NBUF)]
            + [pltpu.SemaphoreType.DMA] * (3 * _NBUF)
        ),
    )
    def k(y32_hbm, idx2_hbm, out_hbm,
          gb0, gb1, gb2, ob0, ob1, ob2, ib0, ib1, ib2,
          gs0, gs1, gs2, os0, os1, os2, is0, is1, is2):
        gbufs = (gb0, gb1, gb2)
        obufs = (ob0, ob1, ob2)
        ibufs = (ib0, ib1, ib2)
        gsem = (gs0, gs1, gs2)
        osem = (os0, os1, os2)
        isem = (is0, is1, is2)
        wid = lax.axis_index("s") * info.num_cores + lax.axis_index("c")
        cbase = wid * nch

        def fire_idx(j, b):
            pltpu.async_copy(
                idx2_hbm.at[pl.ds((cbase + j) * 2 * _C, 2 * _C)],
                ibufs[b], isem[b])

        def wait_idx(j, b):
            pltpu.make_async_copy(
                idx2_hbm.at[pl.ds((cbase + j) * 2 * _C, 2 * _C)],
                ibufs[b], isem[b]).wait()

        def fire_gather(b):
            pltpu.async_copy(y32_hbm.at[ibufs[b]], gbufs[b], gsem[b])

        def wait_gather(b):
            pltpu.make_async_copy(
                y32_hbm.at[ibufs[b]], gbufs[b], gsem[b]).wait()

        def fire_out(j, b):
            pltpu.async_copy(
                obufs[b], out_hbm.at[pl.ds((cbase + j) * _C, _C)], osem[b])

        def wait_out(j, b):
            pltpu.make_async_copy(
                obufs[b], out_hbm.at[pl.ds((cbase + j) * _C, _C)],
                osem[b]).wait()

        for b in range(_NBUF):
            fire_idx(b, b)
        for b in range(2):
            wait_idx(b, b)
            fire_gather(b)

        def slot(j, b):
            b2 = (b + 2) % _NBUF

            @pl.when(j < nch)
            def _process():
                wait_gather(b)

                @pl.when(j + _NBUF < nch)
                def _():
                    fire_idx(j + _NBUF, b)

                @pl.when(j + 2 < nch)
                def _():
                    wait_idx(j + 2, b2)
                    fire_gather(b2)

                @pl.when(j >= _NBUF)
                def _():
                    wait_out(j - _NBUF, b)

                @plsc.parallel_loop(0, dw // _LANES, unroll=4)
                def _add_grp(g):
                    sl = pl.ds(g * _LANES, _LANES)
                    slo = pl.ds(g * 2 * _LANES, _LANES)
                    shi = pl.ds(g * 2 * _LANES + _LANES, _LANES)
                    hi_mask = jnp.int32(-65536)
                    bc = lambda v: lax.bitcast_convert_type(v, jnp.float32)
                    for i in range(_C):
                        av = gbufs[b][i, sl]
                        cv = gbufs[b][_C + i, sl]
                        obufs[b][i, slo] = bc(av << 16) + bc(cv << 16)
                        obufs[b][i, shi] = (
                            bc(av & hi_mask) + bc(cv & hi_mask))
                fire_out(j, b)

        def outer(g, _):
            j0 = g * _NBUF
            for b in range(_NBUF):
                slot(j0 + b, b)
            return 0

        lax.fori_loop(0, n_outer, outer, 0)

        for jj in range(nch - _NBUF, nch):
            wait_out(jj, jj % _NBUF)

    return k(y32, idx2)


def kernel(x, edge_index, W, b):
    n = x.shape[0]
    e = edge_index.shape[1]
    d_out = W.shape[1]
    row = jnp.clip(edge_index[0].astype(jnp.int32), 0, n - 1)
    col = jnp.clip(edge_index[1].astype(jnp.int32), 0, n - 1)
    idx2 = jnp.stack(
        [row.reshape(e // _C, _C), col.reshape(e // _C, _C)], axis=1
    ).reshape(-1)
    cols = jnp.arange(d_out).reshape(d_out // 32, 2, 16)
    lo_cols = cols[:, 0, :].reshape(-1)
    hi_cols = cols[:, 1, :].reshape(-1)
    bh = 0.5 * b
    y32 = _matmul_bias_packed(
        x, W[:, lo_cols], W[:, hi_cols],
        bh[lo_cols].reshape(1, -1).astype(jnp.float32),
        bh[hi_cols].reshape(1, -1).astype(jnp.float32))
    return _gather_add_sc(y32, idx2, e)


# speedup vs baseline: 1.1064x; 1.1064x over previous
import functools

import jax
import jax.numpy as jnp
from jax import lax
from jax.experimental import pallas as pl
from jax.experimental.pallas import tpu as pltpu
from jax.experimental.pallas import tpu_sc as plsc

_LANES = 16


def _mm_body(x_ref, wlo_ref, whi_ref, blo_ref, bhi_ref, y32_ref):
    s_lo = (
        jnp.dot(x_ref[...], wlo_ref[...], preferred_element_type=jnp.float32)
        + blo_ref[...]
    )
    s_hi = (
        jnp.dot(x_ref[...], whi_ref[...], preferred_element_type=jnp.float32)
        + bhi_ref[...]
    )
    u_lo = lax.bitcast_convert_type(
        s_lo.astype(jnp.bfloat16), jnp.uint16).astype(jnp.int32)
    u_hi = lax.bitcast_convert_type(
        s_hi.astype(jnp.bfloat16), jnp.uint16).astype(jnp.int32)
    y32_ref[...] = u_lo | (u_hi << 16)


def _matmul_bias_packed(x, w_lo, w_hi, b_lo, b_hi):
    n, d_in = x.shape
    dw = w_lo.shape[1]
    bn = 2000
    assert n % bn == 0
    return pl.pallas_call(
        _mm_body,
        grid=(n // bn,),
        in_specs=[
            pl.BlockSpec((bn, d_in), lambda i: (i, 0)),
            pl.BlockSpec((d_in, dw), lambda i: (0, 0)),
            pl.BlockSpec((d_in, dw), lambda i: (0, 0)),
            pl.BlockSpec((1, dw), lambda i: (0, 0)),
            pl.BlockSpec((1, dw), lambda i: (0, 0)),
        ],
        out_specs=pl.BlockSpec((bn, dw), lambda i: (i, 0)),
        out_shape=jax.ShapeDtypeStruct((n, dw), jnp.int32),
    )(x, w_lo, w_hi, b_lo, b_hi)


_C = 40
_NBUF = 3


def _gather_add_sc(y32, idx2, e):
    n, dw = y32.shape
    d = 2 * dw
    info = plsc.get_sparse_core_info()
    nw = info.num_cores * info.num_subcores
    assert e % (nw * _C) == 0
    nch = e // (nw * _C)
    n_outer = (nch + _NBUF - 1) // _NBUF
    mesh = plsc.VectorSubcoreMesh(core_axis_name="c", subcore_axis_name="s")

    @functools.partial(
        pl.kernel,
        mesh=mesh,
        out_type=jax.ShapeDtypeStruct((e, d), jnp.float32),
        scratch_types=(
            [pltpu.VMEM((2 * _C, dw), jnp.int32) for _ in range(_NBUF)]
            + [pltpu.VMEM((_C, d), jnp.float32) for _ in range(_NBUF)]
            + [pltpu.VMEM((2 * _C,), jnp.int32) for _ in range(_NBUF)]
            + [pltpu.SemaphoreType.DMA] * (3 * _NBUF)
        ),
    )
    def k(y32_hbm, idx2_hbm, out_hbm,
          gb0, gb1, gb2, ob0, ob1, ob2, ib0, ib1, ib2,
          gs0, gs1, gs2, os0, os1, os2, is0, is1, is2):
        gbufs = (gb0, gb1, gb2)
        obufs = (ob0, ob1, ob2)
        ibufs = (ib0, ib1, ib2)
        gsem = (gs0, gs1, gs2)
        osem = (os0, os1, os2)
        isem = (is0, is1, is2)
        wid = lax.axis_index("s") * info.num_cores + lax.axis_index("c")
        cbase = wid * nch

        def fire_idx(j, b):
            pltpu.async_copy(
                idx2_hbm.at[pl.ds((cbase + j) * 2 * _C, 2 * _C)],
                ibufs[b], isem[b])

        def wait_idx(j, b):
            pltpu.make_async_copy(
                idx2_hbm.at[pl.ds((cbase + j) * 2 * _C, 2 * _C)],
                ibufs[b], isem[b]).wait()

        def fire_gather(b):
            pltpu.async_copy(y32_hbm.at[ibufs[b]], gbufs[b], gsem[b])

        def wait_gather(b):
            pltpu.make_async_copy(
                y32_hbm.at[ibufs[b]], gbufs[b], gsem[b]).wait()

        def fire_out(j, b):
            pltpu.async_copy(
                obufs[b], out_hbm.at[pl.ds((cbase + j) * _C, _C)], osem[b])

        def wait_out(j, b):
            pltpu.make_async_copy(
                obufs[b], out_hbm.at[pl.ds((cbase + j) * _C, _C)],
                osem[b]).wait()

        for b in range(_NBUF):
            fire_idx(b, b)
        for b in range(2):
            wait_idx(b, b)
            fire_gather(b)

        def slot(j, b):
            b2 = (b + 2) % _NBUF

            @pl.when(j < nch)
            def _process():
                wait_gather(b)

                @pl.when(j + _NBUF < nch)
                def _():
                    fire_idx(j + _NBUF, b)

                @pl.when(j + 2 < nch)
                def _():
                    wait_idx(j + 2, b2)
                    fire_gather(b2)

                @pl.when(j >= _NBUF)
                def _():
                    wait_out(j - _NBUF, b)

                @plsc.parallel_loop(0, dw // _LANES, unroll=2)
                def _add_grp(g):
                    sl = pl.ds(g * _LANES, _LANES)
                    slo = pl.ds(g * 2 * _LANES, _LANES)
                    shi = pl.ds(g * 2 * _LANES + _LANES, _LANES)
                    hi_mask = jnp.int32(-65536)
                    bc = lambda v: lax.bitcast_convert_type(v, jnp.float32)
                    for i in range(_C):
                        av = gbufs[b][i, sl]
                        cv = gbufs[b][_C + i, sl]
                        obufs[b][i, slo] = bc(av << 16) + bc(cv << 16)
                        obufs[b][i, shi] = (
                            bc(av & hi_mask) + bc(cv & hi_mask))
                fire_out(j, b)

        def outer(g, _):
            j0 = g * _NBUF
            for b in range(_NBUF):
                slot(j0 + b, b)
            return 0

        lax.fori_loop(0, n_outer, outer, 0)

        for jj in range(nch - _NBUF, nch):
            wait_out(jj, jj % _NBUF)

    return k(y32, idx2)


def kernel(x, edge_index, W, b):
    n = x.shape[0]
    e = edge_index.shape[1]
    d_out = W.shape[1]
    row = jnp.clip(edge_index[0].astype(jnp.int32), 0, n - 1)
    col = jnp.clip(edge_index[1].astype(jnp.int32), 0, n - 1)
    idx2 = jnp.stack(
        [row.reshape(e // _C, _C), col.reshape(e // _C, _C)], axis=1
    ).reshape(-1)
    cols = jnp.arange(d_out).reshape(d_out // 32, 2, 16)
    lo_cols = cols[:, 0, :].reshape(-1)
    hi_cols = cols[:, 1, :].reshape(-1)
    bh = 0.5 * b
    y32 = _matmul_bias_packed(
        x, W[:, lo_cols], W[:, hi_cols],
        bh[lo_cols].reshape(1, -1).astype(jnp.float32),
        bh[hi_cols].reshape(1, -1).astype(jnp.float32))
    return _gather_add_sc(y32, idx2, e)


# speedup vs baseline: 1.2227x; 1.1051x over previous
import functools

import jax
import jax.numpy as jnp
from jax import lax
from jax.experimental import pallas as pl
from jax.experimental.pallas import tpu as pltpu
from jax.experimental.pallas import tpu_sc as plsc

_LANES = 16


def _mm_body(x_ref, wlo_ref, whi_ref, blo_ref, bhi_ref, y32_ref):
    s_lo = (
        jnp.dot(x_ref[...], wlo_ref[...], preferred_element_type=jnp.float32)
        + blo_ref[...]
    )
    s_hi = (
        jnp.dot(x_ref[...], whi_ref[...], preferred_element_type=jnp.float32)
        + bhi_ref[...]
    )
    u_lo = lax.bitcast_convert_type(
        s_lo.astype(jnp.bfloat16), jnp.uint16).astype(jnp.int32)
    u_hi = lax.bitcast_convert_type(
        s_hi.astype(jnp.bfloat16), jnp.uint16).astype(jnp.int32)
    y32_ref[...] = u_lo | (u_hi << 16)


def _matmul_bias_packed(x, w_lo, w_hi, b_lo, b_hi):
    n, d_in = x.shape
    dw = w_lo.shape[1]
    bn = 2000
    assert n % bn == 0
    return pl.pallas_call(
        _mm_body,
        grid=(n // bn,),
        in_specs=[
            pl.BlockSpec((bn, d_in), lambda i: (i, 0)),
            pl.BlockSpec((d_in, dw), lambda i: (0, 0)),
            pl.BlockSpec((d_in, dw), lambda i: (0, 0)),
            pl.BlockSpec((1, dw), lambda i: (0, 0)),
            pl.BlockSpec((1, dw), lambda i: (0, 0)),
        ],
        out_specs=pl.BlockSpec((bn, dw), lambda i: (i, 0)),
        out_shape=jax.ShapeDtypeStruct((n, dw), jnp.int32),
    )(x, w_lo, w_hi, b_lo, b_hi)


_C = 40
_NBUF = 3


def _gather_add_sc(y32, idx2, e):
    n, dw = y32.shape
    d = 2 * dw
    info = plsc.get_sparse_core_info()
    nw = info.num_cores * info.num_subcores
    assert e % (nw * _C) == 0
    nch = e // (nw * _C)
    n_outer = (nch + _NBUF - 1) // _NBUF
    mesh = plsc.VectorSubcoreMesh(core_axis_name="c", subcore_axis_name="s")

    @functools.partial(
        pl.kernel,
        mesh=mesh,
        out_type=jax.ShapeDtypeStruct((e, d), jnp.float32),
        scratch_types=(
            [pltpu.VMEM((2 * _C, dw), jnp.int32) for _ in range(_NBUF)]
            + [pltpu.VMEM((_C, d), jnp.float32) for _ in range(_NBUF)]
            + [pltpu.VMEM((2 * _C,), jnp.int32) for _ in range(_NBUF)]
            + [pltpu.SemaphoreType.DMA] * (3 * _NBUF)
        ),
    )
    def k(y32_hbm, idx2_hbm, out_hbm,
          gb0, gb1, gb2, ob0, ob1, ob2, ib0, ib1, ib2,
          gs0, gs1, gs2, os0, os1, os2, is0, is1, is2):
        gbufs = (gb0, gb1, gb2)
        obufs = (ob0, ob1, ob2)
        ibufs = (ib0, ib1, ib2)
        gsem = (gs0, gs1, gs2)
        osem = (os0, os1, os2)
        isem = (is0, is1, is2)
        wid = lax.axis_index("s") * info.num_cores + lax.axis_index("c")
        cbase = wid * nch

        def fire_idx(j, b):
            pltpu.async_copy(
                idx2_hbm.at[pl.ds((cbase + j) * 2 * _C, 2 * _C)],
                ibufs[b], isem[b])

        def wait_idx(j, b):
            pltpu.make_async_copy(
                idx2_hbm.at[pl.ds((cbase + j) * 2 * _C, 2 * _C)],
                ibufs[b], isem[b]).wait()

        def fire_gather(b):
            pltpu.async_copy(y32_hbm.at[ibufs[b]], gbufs[b], gsem[b])

        def wait_gather(b):
            pltpu.make_async_copy(
                y32_hbm.at[ibufs[b]], gbufs[b], gsem[b]).wait()

        def fire_out(j, b):
            pltpu.async_copy(
                obufs[b], out_hbm.at[pl.ds((cbase + j) * _C, _C)], osem[b])

        def wait_out(j, b):
            pltpu.make_async_copy(
                obufs[b], out_hbm.at[pl.ds((cbase + j) * _C, _C)],
                osem[b]).wait()

        for b in range(_NBUF):
            fire_idx(b, b)
        for b in range(2):
            wait_idx(b, b)
            fire_gather(b)

        def slot(j, b):
            b2 = (b + 2) % _NBUF

            @pl.when(j < nch)
            def _process():
                wait_gather(b)

                @pl.when(j + _NBUF < nch)
                def _():
                    fire_idx(j + _NBUF, b)

                @pl.when(j + 2 < nch)
                def _():
                    wait_idx(j + 2, b2)
                    fire_gather(b2)

                @pl.when(j >= _NBUF)
                def _():
                    wait_out(j - _NBUF, b)

                @plsc.parallel_loop(0, _C, unroll=2)
                def _add_row(i):
                    hi_mask = jnp.int32(-65536)
                    bc = lambda v: lax.bitcast_convert_type(v, jnp.float32)
                    for g in range(dw // _LANES):
                        sl = pl.ds(g * _LANES, _LANES)
                        slo = pl.ds(g * 2 * _LANES, _LANES)
                        shi = pl.ds(g * 2 * _LANES + _LANES, _LANES)
                        av = gbufs[b][i, sl]
                        cv = gbufs[b][_C + i, sl]
                        obufs[b][i, slo] = bc(av << 16) + bc(cv << 16)
                        obufs[b][i, shi] = (
                            bc(av & hi_mask) + bc(cv & hi_mask))
                fire_out(j, b)

        def outer(g, _):
            j0 = g * _NBUF
            for b in range(_NBUF):
                slot(j0 + b, b)
            return 0

        lax.fori_loop(0, n_outer, outer, 0)

        for jj in range(nch - _NBUF, nch):
            wait_out(jj, jj % _NBUF)

    return k(y32, idx2)


def kernel(x, edge_index, W, b):
    n = x.shape[0]
    e = edge_index.shape[1]
    d_out = W.shape[1]
    row = jnp.clip(edge_index[0].astype(jnp.int32), 0, n - 1)
    col = jnp.clip(edge_index[1].astype(jnp.int32), 0, n - 1)
    idx2 = jnp.stack(
        [row.reshape(e // _C, _C), col.reshape(e // _C, _C)], axis=1
    ).reshape(-1)
    cols = jnp.arange(d_out).reshape(d_out // 32, 2, 16)
    lo_cols = cols[:, 0, :].reshape(-1)
    hi_cols = cols[:, 1, :].reshape(-1)
    bh = 0.5 * b
    y32 = _matmul_bias_packed(
        x, W[:, lo_cols], W[:, hi_cols],
        bh[lo_cols].reshape(1, -1).astype(jnp.float32),
        bh[hi_cols].reshape(1, -1).astype(jnp.float32))
    return _gather_add_sc(y32, idx2, e)
